# Initial kernel scaffold; baseline (speedup 1.0000x reference)
import jax
import jax.numpy as jnp
from jax.experimental import pallas as pl


def kernel(x, card_buffer):
    raise NotImplementedError("write your pallas kernel here")


# speedup vs baseline: 2.8126x; 2.8126x over previous
import functools

import jax
import jax.numpy as jnp
from jax import lax
from jax.experimental import pallas as pl
from jax.experimental.pallas import tpu as pltpu
from jax.experimental.pallas import tpu_sc as plsc

BATCH = 16384
IN_DIM = 128
EMB_DIM = 18
RMIN = 64
RMAX = 71
OUT_W = IN_DIM * EMB_DIM
NEMB = (RMAX - RMIN) * EMB_DIM

L = 16
NC = 2
NS = 16
NW = NC * NS
ROWS_PER_W = BATCH // NW
CHUNK = 32
NCHUNKS = ROWS_PER_W // CHUNK
NBV = IN_DIM // L
NEG = (NEMB + L - 1) // L


def _sc_expand(x_hbm, cb_hbm, out_hbm, x_v, cb_v, out_v):
    wid = lax.axis_index("s") * NC + lax.axis_index("c")
    pltpu.sync_copy(cb_hbm, cb_v)

    iota = lax.iota(jnp.int32, L)
    base18 = iota * EMB_DIM
    gconst = []
    for g in range(NEG):
        pv = iota + g * L
        cv = pv // EMB_DIM
        jv = pv - cv * EMB_DIM
        gconst.append((pv, cv, jv))
    mask_last = (iota + (NEG - 1) * L) < NEMB

    row0 = wid * ROWS_PER_W

    def chunk_body(c, carry):
        base = row0 + c * CHUNK
        pltpu.sync_copy(x_hbm.at[pl.ds(base * IN_DIM, CHUNK * IN_DIM)], x_v)

        def row_body(r, rcarry):
            roff = jnp.full((L,), r * OUT_W, jnp.int32)
            rb = roff + base18
            for v in range(NBV):
                xv = x_v[pl.ds(r * IN_DIM + v * L, L)]
                for j in range(EMB_DIM):
                    plsc.store_scatter(out_v, [rb + (v * L * EMB_DIM + j)], xv)
            for g in range(NEG):
                pv, cv, jv = gconst[g]
                cards = plsc.load_gather(x_v, [cv + (r * IN_DIM + RMIN)])
                ci = cards.astype(jnp.int32) * EMB_DIM + jv
                emb = plsc.load_gather(cb_v, [ci])
                dst = roff + (RMIN * EMB_DIM) + pv
                if g == NEG - 1:
                    plsc.store_scatter(out_v, [dst], emb, mask=mask_last)
                else:
                    plsc.store_scatter(out_v, [dst], emb)
            return rcarry

        lax.fori_loop(0, CHUNK, row_body, 0)
        pltpu.sync_copy(out_v, out_hbm.at[pl.ds(base * OUT_W, CHUNK * OUT_W)])
        return carry

    lax.fori_loop(0, NCHUNKS, chunk_body, 0)


@jax.jit
def _run(xf, cbf):
    fn = functools.partial(
        pl.kernel,
        mesh=plsc.VectorSubcoreMesh(core_axis_name="c", subcore_axis_name="s"),
        compiler_params=pltpu.CompilerParams(needs_layout_passes=False),
        out_type=jax.ShapeDtypeStruct((BATCH * OUT_W,), jnp.float32),
        scratch_types=[
            pltpu.VMEM((CHUNK * IN_DIM,), jnp.float32),
            pltpu.VMEM((52 * EMB_DIM,), jnp.float32),
            pltpu.VMEM((CHUNK * OUT_W,), jnp.float32),
        ],
    )(_sc_expand)
    return fn(xf, cbf)


def kernel(x, card_buffer):
    if x.ndim == 3:
        x = x[:, 0, :]
    xf = x.reshape(BATCH * IN_DIM)
    cbf = card_buffer.reshape(52 * EMB_DIM)
    out = _run(xf, cbf)
    return out.reshape(BATCH, IN_DIM, EMB_DIM)


# speedup vs baseline: 3.0787x; 1.0946x over previous
import functools

import jax
import jax.numpy as jnp
from jax import lax
from jax.experimental import pallas as pl
from jax.experimental.pallas import tpu as pltpu
from jax.experimental.pallas import tpu_sc as plsc

BATCH = 16384
IN_DIM = 128
EMB_DIM = 18
RMIN = 64
RMAX = 71
OUT_W = IN_DIM * EMB_DIM
NEMB = (RMAX - RMIN) * EMB_DIM

L = 16
NC = 2
NS = 16
NW = NC * NS
ROWS_PER_W = BATCH // NW
CHUNK = 32
NCHUNKS = ROWS_PER_W // CHUNK
NBV = IN_DIM // L
BLK = L * EMB_DIM
EMB_BLK = RMIN // L
NEV = NEMB // L
EMB_TAIL = NEMB - NEV * L


def _sc_expand(x_hbm, cb_hbm, out_hbm, x_v, cb_v, out_v):
    wid = lax.axis_index("s") * NC + lax.axis_index("c")
    pltpu.sync_copy(cb_hbm, cb_v)

    iota = lax.iota(jnp.int32, L)
    perms = []
    jmods = []
    for t in range(EMB_DIM):
        q = iota + t * L
        perms.append(q // EMB_DIM)
        jmods.append(q - (q // EMB_DIM) * EMB_DIM)
    mask_tail = iota < EMB_TAIL

    row0 = wid * ROWS_PER_W

    def chunk_body(c, carry):
        base = row0 + c * CHUNK
        pltpu.sync_copy(x_hbm.at[pl.ds(base * IN_DIM, CHUNK * IN_DIM)], x_v)

        def row_body(r, rcarry):
            xr = r * IN_DIM
            orow = r * OUT_W
            for v in range(NBV):
                xv = x_v[pl.ds(xr + v * L, L)]
                for t in range(EMB_DIM):
                    bc = jnp.take_along_axis(
                        xv, perms[t], axis=0, mode="promise_in_bounds"
                    )
                    if v == EMB_BLK and t <= NEV:
                        ci = bc.astype(jnp.int32) * EMB_DIM + jmods[t]
                        emb = plsc.load_gather(cb_v, [ci])
                        if t == NEV:
                            val = jnp.where(mask_tail, emb, bc)
                        else:
                            val = emb
                    else:
                        val = bc
                    out_v[pl.ds(orow + v * BLK + t * L, L)] = val
            return rcarry

        lax.fori_loop(0, CHUNK, row_body, 0)
        pltpu.sync_copy(out_v, out_hbm.at[pl.ds(base * OUT_W, CHUNK * OUT_W)])
        return carry

    lax.fori_loop(0, NCHUNKS, chunk_body, 0)


@jax.jit
def _run(xf, cbf):
    fn = functools.partial(
        pl.kernel,
        mesh=plsc.VectorSubcoreMesh(core_axis_name="c", subcore_axis_name="s"),
        compiler_params=pltpu.CompilerParams(needs_layout_passes=False),
        out_type=jax.ShapeDtypeStruct((BATCH * OUT_W,), jnp.float32),
        scratch_types=[
            pltpu.VMEM((CHUNK * IN_DIM,), jnp.float32),
            pltpu.VMEM((52 * EMB_DIM,), jnp.float32),
            pltpu.VMEM((CHUNK * OUT_W,), jnp.float32),
        ],
    )(_sc_expand)
    return fn(xf, cbf)


def kernel(x, card_buffer):
    if x.ndim == 3:
        x = x[:, 0, :]
    xf = x.reshape(BATCH * IN_DIM)
    cbf = card_buffer.reshape(52 * EMB_DIM)
    out = _run(xf, cbf)
    return out.reshape(BATCH, IN_DIM, EMB_DIM)


# speedup vs baseline: 28.8366x; 9.3664x over previous
import functools

import jax
import jax.numpy as jnp
from jax import lax
from jax.experimental import pallas as pl
from jax.experimental.pallas import tpu as pltpu
from jax.experimental.pallas import tpu_sc as plsc

BATCH = 16384
IN_DIM = 128
EMB_DIM = 18
RMIN = 64
RMAX = 71
NPATCH = RMAX - RMIN

L = 16
NC = 2
NS = 16
NW = NC * NS
ROWS_PER_W = BATCH // NW
CHUNK = 64
NCHUNKS = ROWS_PER_W // CHUNK


def _sc_planes(x_hbm, cb_hbm, out_hbm, x_v, cb_v, patch_v, sem):
    wid = lax.axis_index("s") * NC + lax.axis_index("c")
    pltpu.sync_copy(cb_hbm, cb_v)

    iota = lax.iota(jnp.int32, L)
    mask_patch = iota < NPATCH

    row0 = wid * ROWS_PER_W

    def chunk_body(c, carry):
        base = row0 + c * CHUNK
        pltpu.sync_copy(x_hbm.at[pl.ds(base, CHUNK)], x_v)

        copies = []
        for j in range(EMB_DIM):
            copies.append(
                pltpu.async_copy(
                    x_v.at[:, pl.ds(0, RMIN)],
                    out_hbm.at[j, pl.ds(base, CHUNK), pl.ds(0, RMIN)],
                    sem,
                )
            )
            copies.append(
                pltpu.async_copy(
                    x_v.at[:, pl.ds(RMIN + L, IN_DIM - RMIN - L)],
                    out_hbm.at[
                        j, pl.ds(base, CHUNK), pl.ds(RMIN + L, IN_DIM - RMIN - L)
                    ],
                    sem,
                )
            )

        def row_body(r, rcarry):
            xv = x_v[r, pl.ds(RMIN, L)]
            ci = xv.astype(jnp.int32) * EMB_DIM
            for j in range(EMB_DIM):
                emb = plsc.load_gather(cb_v, [ci + j])
                patch_v[j, r] = jnp.where(mask_patch, emb, xv)
            return rcarry

        lax.fori_loop(0, CHUNK, row_body, 0)

        for j in range(EMB_DIM):
            copies.append(
                pltpu.async_copy(
                    patch_v.at[j],
                    out_hbm.at[j, pl.ds(base, CHUNK), pl.ds(RMIN, L)],
                    sem,
                )
            )
        for cp in copies:
            cp.wait()
        return carry

    lax.fori_loop(0, NCHUNKS, chunk_body, 0)


@jax.jit
def _run(x, cbf):
    fn = functools.partial(
        pl.kernel,
        mesh=plsc.VectorSubcoreMesh(core_axis_name="c", subcore_axis_name="s"),
        compiler_params=pltpu.CompilerParams(
            needs_layout_passes=False, use_tc_tiling_on_sc=False
        ),
        out_type=jax.ShapeDtypeStruct((EMB_DIM, BATCH, IN_DIM), jnp.float32),
        scratch_types=[
            pltpu.VMEM((CHUNK, IN_DIM), jnp.float32),
            pltpu.VMEM((52 * EMB_DIM,), jnp.float32),
            pltpu.VMEM((EMB_DIM, CHUNK, L), jnp.float32),
            pltpu.SemaphoreType.DMA,
        ],
    )(_sc_planes)
    return fn(x, cbf)


def kernel(x, card_buffer):
    if x.ndim == 3:
        x = x[:, 0, :]
    cbf = card_buffer.reshape(52 * EMB_DIM)
    out = _run(x, cbf)
    return out.transpose(1, 2, 0)


# speedup vs baseline: 29.2741x; 1.0152x over previous
import functools

import jax
import jax.numpy as jnp
from jax import lax
from jax.experimental import pallas as pl
from jax.experimental.pallas import tpu as pltpu
from jax.experimental.pallas import tpu_sc as plsc

BATCH = 16384
IN_DIM = 128
EMB_DIM = 18
RMIN = 64
RMAX = 71
NPATCH = RMAX - RMIN

L = 16
NC = 2
NS = 16
NW = NC * NS
ROWS_PER_W = BATCH // NW
CHUNK = 128
NCHUNKS = ROWS_PER_W // CHUNK


def _sc_planes(x_hbm, cb_hbm, out_hbm, x_v, cb_v, patch_v, sem):
    wid = lax.axis_index("s") * NC + lax.axis_index("c")
    pltpu.sync_copy(cb_hbm, cb_v)

    iota = lax.iota(jnp.int32, L)
    mask_patch = iota < NPATCH

    row0 = wid * ROWS_PER_W

    def chunk_body(c, carry):
        base = row0 + c * CHUNK
        pltpu.sync_copy(x_hbm.at[pl.ds(base, CHUNK)], x_v)

        copies = []
        for j in range(EMB_DIM):
            copies.append(
                pltpu.async_copy(
                    x_v.at[:, pl.ds(0, RMIN)],
                    out_hbm.at[j, pl.ds(base, CHUNK), pl.ds(0, RMIN)],
                    sem,
                )
            )
            copies.append(
                pltpu.async_copy(
                    x_v.at[:, pl.ds(RMIN + L, IN_DIM - RMIN - L)],
                    out_hbm.at[
                        j, pl.ds(base, CHUNK), pl.ds(RMIN + L, IN_DIM - RMIN - L)
                    ],
                    sem,
                )
            )

        def row_body(r, rcarry):
            xv = x_v[r, pl.ds(RMIN, L)]
            ci = xv.astype(jnp.int32) * EMB_DIM
            for j in range(EMB_DIM):
                emb = plsc.load_gather(cb_v, [ci + j])
                patch_v[j, r] = jnp.where(mask_patch, emb, xv)
            return rcarry

        lax.fori_loop(0, CHUNK, row_body, 0)

        copies.append(
            pltpu.async_copy(
                patch_v,
                out_hbm.at[pl.ds(0, EMB_DIM), pl.ds(base, CHUNK), pl.ds(RMIN, L)],
                sem,
            )
        )
        for cp in copies:
            cp.wait()
        return carry

    lax.fori_loop(0, NCHUNKS, chunk_body, 0)


@jax.jit
def _run(x, cbf):
    fn = functools.partial(
        pl.kernel,
        mesh=plsc.VectorSubcoreMesh(core_axis_name="c", subcore_axis_name="s"),
        compiler_params=pltpu.CompilerParams(
            needs_layout_passes=False, use_tc_tiling_on_sc=False
        ),
        out_type=jax.ShapeDtypeStruct((EMB_DIM, BATCH, IN_DIM), jnp.float32),
        scratch_types=[
            pltpu.VMEM((CHUNK, IN_DIM), jnp.float32),
            pltpu.VMEM((52 * EMB_DIM,), jnp.float32),
            pltpu.VMEM((EMB_DIM, CHUNK, L), jnp.float32),
            pltpu.SemaphoreType.DMA,
        ],
    )(_sc_planes)
    return fn(x, cbf)


def kernel(x, card_buffer):
    if x.ndim == 3:
        x = x[:, 0, :]
    cbf = card_buffer.reshape(52 * EMB_DIM)
    out = _run(x, cbf)
    return out.transpose(1, 2, 0)


# speedup vs baseline: 58.4668x; 1.9972x over previous
import functools

import jax
import jax.numpy as jnp
from jax import lax
from jax.experimental import pallas as pl
from jax.experimental.pallas import tpu as pltpu
from jax.experimental.pallas import tpu_sc as plsc

BATCH = 16384
IN_DIM = 128
EMB_DIM = 18
RMIN = 64
RMAX = 71
NPATCH = RMAX - RMIN

L = 16
NC = 2
NS = 16
NW = NC * NS
ROWS_PER_W = BATCH // NW
CHUNK = 128
NCHUNKS = ROWS_PER_W // CHUNK


def _sc_planes(x_hbm, cb_hbm, out_hbm, x_v, cb_v, patch_v, sem):
    wid = lax.axis_index("s") * NC + lax.axis_index("c")
    pltpu.sync_copy(cb_hbm, cb_v)

    iota = lax.iota(jnp.int32, L)
    mask_patch = iota < NPATCH

    row0 = wid * ROWS_PER_W

    def chunk_body(c, carry):
        base = row0 + c * CHUNK
        pltpu.sync_copy(x_hbm.at[pl.ds(base, CHUNK)], x_v)

        copies = []
        for j in range(EMB_DIM):
            copies.append(
                pltpu.async_copy(
                    x_v,
                    out_hbm.at[j, pl.ds(base, CHUNK)],
                    sem,
                )
            )

        def row_body(r, rcarry):
            xv = x_v[r, pl.ds(RMIN, L)]
            ci = xv.astype(jnp.int32) * EMB_DIM
            for j in range(EMB_DIM):
                emb = plsc.load_gather(cb_v, [ci + j])
                patch_v[j, r] = jnp.where(mask_patch, emb, xv)
            return rcarry

        lax.fori_loop(0, CHUNK, row_body, 0)

        for cp in copies:
            cp.wait()
        pltpu.async_copy(
            patch_v,
            out_hbm.at[pl.ds(0, EMB_DIM), pl.ds(base, CHUNK), pl.ds(RMIN, L)],
            sem,
        ).wait()
        return carry

    lax.fori_loop(0, NCHUNKS, chunk_body, 0)


@jax.jit
def _run(x, cbf):
    fn = functools.partial(
        pl.kernel,
        mesh=plsc.VectorSubcoreMesh(core_axis_name="c", subcore_axis_name="s"),
        compiler_params=pltpu.CompilerParams(
            needs_layout_passes=False, use_tc_tiling_on_sc=False
        ),
        out_type=jax.ShapeDtypeStruct((EMB_DIM, BATCH, IN_DIM), jnp.float32),
        scratch_types=[
            pltpu.VMEM((CHUNK, IN_DIM), jnp.float32),
            pltpu.VMEM((52 * EMB_DIM,), jnp.float32),
            pltpu.VMEM((EMB_DIM, CHUNK, L), jnp.float32),
            pltpu.SemaphoreType.DMA,
        ],
    )(_sc_planes)
    return fn(x, cbf)


def kernel(x, card_buffer):
    if x.ndim == 3:
        x = x[:, 0, :]
    cbf = card_buffer.reshape(52 * EMB_DIM)
    out = _run(x, cbf)
    return out.transpose(1, 2, 0)


# speedup vs baseline: 60.2591x; 1.0307x over previous
import functools

import jax
import jax.numpy as jnp
from jax import lax
from jax.experimental import pallas as pl
from jax.experimental.pallas import tpu as pltpu
from jax.experimental.pallas import tpu_sc as plsc

BATCH = 16384
IN_DIM = 128
EMB_DIM = 18
RMIN = 64
RMAX = 71
NPATCH = RMAX - RMIN

L = 16
NC = 2
NS = 16
NW = NC * NS
ROWS_PER_W = BATCH // NW
CHUNK = 256
NCHUNKS = ROWS_PER_W // CHUNK


def _sc_planes(x_hbm, cb_hbm, out_hbm, x_v, cb_v, patch_v, sem):
    wid = lax.axis_index("s") * NC + lax.axis_index("c")
    pltpu.sync_copy(cb_hbm, cb_v)

    iota = lax.iota(jnp.int32, L)
    mask_patch = iota < NPATCH

    row0 = wid * ROWS_PER_W

    def chunk_body(c, carry):
        base = row0 + c * CHUNK
        pltpu.sync_copy(x_hbm.at[pl.ds(base, CHUNK)], x_v)

        copies = []
        for j in range(EMB_DIM):
            copies.append(
                pltpu.async_copy(
                    x_v,
                    out_hbm.at[j, pl.ds(base, CHUNK)],
                    sem,
                )
            )

        def row_body(r, rcarry):
            xv = x_v[r, pl.ds(RMIN, L)]
            ci = xv.astype(jnp.int32) * EMB_DIM
            for j in range(EMB_DIM):
                emb = plsc.load_gather(cb_v, [ci + j])
                patch_v[j, r] = jnp.where(mask_patch, emb, xv)
            return rcarry

        lax.fori_loop(0, CHUNK, row_body, 0)

        for cp in copies:
            cp.wait()
        pltpu.async_copy(
            patch_v,
            out_hbm.at[pl.ds(0, EMB_DIM), pl.ds(base, CHUNK), pl.ds(RMIN, L)],
            sem,
        ).wait()
        return carry

    lax.fori_loop(0, NCHUNKS, chunk_body, 0)


@jax.jit
def _run(x, cbf):
    fn = functools.partial(
        pl.kernel,
        mesh=plsc.VectorSubcoreMesh(core_axis_name="c", subcore_axis_name="s"),
        compiler_params=pltpu.CompilerParams(
            needs_layout_passes=False, use_tc_tiling_on_sc=False
        ),
        out_type=jax.ShapeDtypeStruct((EMB_DIM, BATCH, IN_DIM), jnp.float32),
        scratch_types=[
            pltpu.VMEM((CHUNK, IN_DIM), jnp.float32),
            pltpu.VMEM((52 * EMB_DIM,), jnp.float32),
            pltpu.VMEM((EMB_DIM, CHUNK, L), jnp.float32),
            pltpu.SemaphoreType.DMA,
        ],
    )(_sc_planes)
    return fn(x, cbf)


def kernel(x, card_buffer):
    if x.ndim == 3:
        x = x[:, 0, :]
    cbf = card_buffer.reshape(52 * EMB_DIM)
    out = _run(x, cbf)
    return out.transpose(1, 2, 0)


# speedup vs baseline: 60.4364x; 1.0029x over previous
import functools

import jax
import jax.numpy as jnp
from jax import lax
from jax.experimental import pallas as pl
from jax.experimental.pallas import tpu as pltpu
from jax.experimental.pallas import tpu_sc as plsc

BATCH = 16384
IN_DIM = 128
EMB_DIM = 18
RMIN = 64
RMAX = 71
NPATCH = RMAX - RMIN

L = 16
NC = 2
NS = 16
NW = NC * NS
ROWS_PER_W = BATCH // NW
CHUNK = 256
NCHUNKS = ROWS_PER_W // CHUNK


def _sc_planes(x_hbm, cb_hbm, out_hbm, x_v, cb_v, patch_v, sem, psem):
    wid = lax.axis_index("s") * NC + lax.axis_index("c")
    pltpu.sync_copy(cb_hbm, cb_v)

    iota = lax.iota(jnp.int32, L)
    mask_patch = iota < NPATCH

    row0 = wid * ROWS_PER_W

    def chunk_body(c, carry):
        base = row0 + c * CHUNK
        pltpu.sync_copy(x_hbm.at[pl.ds(base, CHUNK)], x_v)

        copies = []
        for j in range(EMB_DIM):
            copies.append(
                pltpu.async_copy(
                    x_v,
                    out_hbm.at[j, pl.ds(base, CHUNK)],
                    sem,
                )
            )

        @pl.when(c > 0)
        def _():
            pltpu.make_async_copy(
                patch_v,
                out_hbm.at[
                    pl.ds(0, EMB_DIM), pl.ds(base, CHUNK), pl.ds(RMIN, L)
                ],
                psem,
            ).wait()

        def row_body(r, rcarry):
            xv = x_v[r, pl.ds(RMIN, L)]
            ci = xv.astype(jnp.int32) * EMB_DIM
            for j in range(EMB_DIM):
                emb = plsc.load_gather(cb_v, [ci + j])
                patch_v[j, r] = jnp.where(mask_patch, emb, xv)
            return rcarry

        lax.fori_loop(0, CHUNK, row_body, 0)

        for cp in copies:
            cp.wait()
        pltpu.async_copy(
            patch_v,
            out_hbm.at[pl.ds(0, EMB_DIM), pl.ds(base, CHUNK), pl.ds(RMIN, L)],
            psem,
        )
        return carry

    lax.fori_loop(0, NCHUNKS, chunk_body, 0)
    last = row0 + (NCHUNKS - 1) * CHUNK
    pltpu.make_async_copy(
        patch_v,
        out_hbm.at[pl.ds(0, EMB_DIM), pl.ds(last, CHUNK), pl.ds(RMIN, L)],
        psem,
    ).wait()


@jax.jit
def _run(x, cbf):
    fn = functools.partial(
        pl.kernel,
        mesh=plsc.VectorSubcoreMesh(core_axis_name="c", subcore_axis_name="s"),
        compiler_params=pltpu.CompilerParams(
            needs_layout_passes=False, use_tc_tiling_on_sc=False
        ),
        out_type=jax.ShapeDtypeStruct((EMB_DIM, BATCH, IN_DIM), jnp.float32),
        scratch_types=[
            pltpu.VMEM((CHUNK, IN_DIM), jnp.float32),
            pltpu.VMEM((52 * EMB_DIM,), jnp.float32),
            pltpu.VMEM((EMB_DIM, CHUNK, L), jnp.float32),
            pltpu.SemaphoreType.DMA,
            pltpu.SemaphoreType.DMA,
        ],
    )(_sc_planes)
    return fn(x, cbf)


def kernel(x, card_buffer):
    if x.ndim == 3:
        x = x[:, 0, :]
    cbf = card_buffer.reshape(52 * EMB_DIM)
    out = _run(x, cbf)
    return out.transpose(1, 2, 0)
